# Initial kernel scaffold; baseline (speedup 1.0000x reference)
import jax
import jax.numpy as jnp
from jax.experimental import pallas as pl


def kernel(queries, query_lens, weight):
    raise NotImplementedError("write your pallas kernel here")


# speedup vs baseline: 1.0545x; 1.0545x over previous
import functools
import jax
import jax.numpy as jnp
from jax import lax
from jax.experimental import pallas as pl
from jax.experimental.pallas import tpu as pltpu
from jax.experimental.pallas import tpu_sc as plsc

D = 32
QMAX = 50
NC = 2
NS = 16
NW = NC * NS
SUB = 100
CHUNK = 800
BPC = CHUNK // QMAX
NSUB = CHUNK // SUB


def _build(n_rows, n_batch):
    rows_per_w = n_rows // NW
    chunks_per_w = rows_per_w // CHUNK
    batch_per_w = n_batch // NW
    mesh = plsc.VectorSubcoreMesh(
        core_axis_name="c", subcore_axis_name="s",
        num_cores=NC, num_subcores=NS)

    @functools.partial(
        pl.kernel,
        out_type=jax.ShapeDtypeStruct((n_rows, D), jnp.float32),
        mesh=mesh,
        compiler_params=pltpu.CompilerParams(use_tc_tiling_on_sc=False),
        scratch_types=[
            pltpu.VMEM((batch_per_w,), jnp.int32),
            pltpu.VMEM((NSUB, SUB), jnp.int32),
            pltpu.VMEM((CHUNK, D), jnp.float32),
            pltpu.SemaphoreType.DMA,
        ],
    )
    def emb(q_hbm, lens_hbm, w_hbm, out_hbm, lens_v, idx_v, rows_v, sem):
        wid = lax.axis_index("s") * NC + lax.axis_index("c")
        pltpu.sync_copy(lens_hbm.at[pl.ds(wid * batch_per_w, batch_per_w)],
                        lens_v)
        zero16 = jnp.zeros((16,), jnp.float32)

        def chunk_body(ci, _):
            base = wid * rows_per_w + ci * CHUNK
            pltpu.sync_copy(
                q_hbm.at[pl.ds(pl.multiple_of(base // SUB, 8), NSUB)], idx_v)
            copies = []
            for s in range(NSUB):
                copies.append(pltpu.async_copy(
                    w_hbm.at[idx_v.at[s]],
                    rows_v.at[pl.ds(s * SUB, SUB)], sem))
            for c in copies:
                c.wait()

            lens16 = lens_v[pl.ds(ci * BPC, BPC)]
            for bb in range(BPC):
                ln = lens16[bb]

                def zero_row(r, _, bb=bb):
                    rows_v[bb * QMAX + r, pl.ds(0, 16)] = zero16
                    rows_v[bb * QMAX + r, pl.ds(16, 16)] = zero16
                    return 0
                lax.fori_loop(ln, QMAX, zero_row, 0)

            pltpu.sync_copy(rows_v, out_hbm.at[pl.ds(base, CHUNK)])
            return 0

        lax.fori_loop(0, chunks_per_w, chunk_body, 0)

    return emb


def kernel(queries, query_lens, weight):
    bsz, qmax = queries.shape
    n_rows = bsz * qmax
    q2d = queries.reshape(n_rows // SUB, SUB)
    out = _build(n_rows, bsz)(q2d, query_lens, weight)
    return out.reshape(bsz, qmax, D)


# speedup vs baseline: 2.7310x; 2.5898x over previous
import functools
import jax
import jax.numpy as jnp
from jax import lax
from jax.experimental import pallas as pl
from jax.experimental.pallas import tpu as pltpu
from jax.experimental.pallas import tpu_sc as plsc

D = 32
QMAX = 50
NC = 2
NS = 16
NW = NC * NS
BPW = 512
SUB = 128
GPQ = BPW // SUB
KH = D // 8
BH = BPW // 128
PAD = 129
QROWS = BPW * QMAX // 128


def _build(n_batch):
    mesh = plsc.VectorSubcoreMesh(
        core_axis_name="c", subcore_axis_name="s",
        num_cores=NC, num_subcores=NS)
    nbh_total = n_batch // 128

    @functools.partial(
        pl.kernel,
        out_type=jax.ShapeDtypeStruct((QMAX, KH, nbh_total, 8, 128),
                                      jnp.float32),
        mesh=mesh,
        compiler_params=pltpu.CompilerParams(
            use_tc_tiling_on_sc=False, needs_layout_passes=False),
        scratch_types=[
            pltpu.VMEM((QROWS, 128), jnp.int32),
            pltpu.VMEM((n_batch // NW,), jnp.int32),
            pltpu.VMEM((2, GPQ, SUB), jnp.int32),
            pltpu.VMEM((2, BPW, D), jnp.float32),
            pltpu.VMEM((2, BH, KH, 8, PAD), jnp.float32),
            pltpu.SemaphoreType.DMA,
            pltpu.SemaphoreType.DMA,
        ],
    )
    def emb(q_hbm, lens_hbm, w_hbm, out_hbm,
            qall_v, lens_v, idxq_v, rows_v, stg_v, gsem, osem):
        wid = lax.axis_index("s") * NC + lax.axis_index("c")
        pltpu.sync_copy(
            q_hbm.at[pl.ds(pl.multiple_of(wid * QROWS, 8), QROWS)], qall_v)
        pltpu.sync_copy(lens_hbm.at[pl.ds(wid * BPW, BPW)], lens_v)

        lane = lax.iota(jnp.int32, 16)
        lane_q = lane * QMAX
        kh_lo = lane // 8
        kh_hi = kh_lo + 2
        kl_v = lane % 8

        def build_idx(qi, buf):
            def grp(g, _):
                base = jnp.full((16,), g * (16 * QMAX) + qi, jnp.int32)
                flat = lane_q + base
                src = plsc.load_gather(
                    qall_v, [flat >> 7, flat & 127])
                idxq_v[buf, g // 8, pl.ds((g % 8) * 16, 16)] = src
                return 0
            lax.fori_loop(0, BPW // 16, grp, 0)

        def start_gather(buf):
            for s in range(GPQ):
                pltpu.async_copy(w_hbm.at[idxq_v.at[buf, s]],
                                 rows_v.at[buf, pl.ds(s * SUB, SUB)], gsem)

        def wait_gather(buf):
            for s in range(GPQ):
                pltpu.make_async_copy(w_hbm.at[idxq_v.at[buf, s]],
                                      rows_v.at[buf, pl.ds(s * SUB, SUB)],
                                      gsem).wait()

        def transpose_q(qi, buf):
            def grp(g, _):
                lens16 = lens_v[pl.ds(g * 16, 16)]
                bh = g // 8
                bl0 = (g % 8) * 16
                qi_v = jnp.full((16,), qi, jnp.int32)
                mask16 = jnp.where(qi_v < lens16, 1.0, 0.0).astype(jnp.float32)
                for j in range(16):
                    b = g * 16 + j
                    m_v = jnp.full((16,), mask16[j], jnp.float32)
                    r0 = rows_v[buf, b, pl.ds(0, 16)] * m_v
                    r1 = rows_v[buf, b, pl.ds(16, 16)] * m_v
                    bh_v = jnp.full((16,), bh, jnp.int32)
                    bl_v = jnp.full((16,), bl0 + j, jnp.int32)
                    plsc.store_scatter(stg_v.at[buf],
                                       [bh_v, kh_lo, kl_v, bl_v], r0)
                    plsc.store_scatter(stg_v.at[buf],
                                       [bh_v, kh_hi, kl_v, bl_v], r1)
                return 0
            lax.fori_loop(0, BPW // 16, grp, 0)

        def start_out(qi, buf):
            for bh in range(BH):
                for kh in range(KH):
                    pltpu.async_copy(
                        stg_v.at[buf, bh, kh, :, pl.ds(0, 128)],
                        out_hbm.at[qi, kh, wid * BH + bh], osem)

        def wait_out(buf):
            for bh in range(BH):
                for kh in range(KH):
                    pltpu.make_async_copy(
                        stg_v.at[buf, bh, kh, :, pl.ds(0, 128)],
                        out_hbm.at[0, kh, wid * BH + bh], osem).wait()

        build_idx(0, 0)
        start_gather(0)

        def pair_body(t, _):
            qa = t * 2
            qb = qa + 1
            build_idx(qb, 1)
            wait_gather(0)
            start_gather(1)

            @pl.when(t > 0)
            def _():
                wait_out(0)
            transpose_q(qa, 0)
            start_out(qa, 0)

            @pl.when(t < QMAX // 2 - 1)
            def _():
                build_idx(qa + 2, 0)
            wait_gather(1)

            @pl.when(t < QMAX // 2 - 1)
            def _():
                start_gather(0)

            @pl.when(t > 0)
            def _():
                wait_out(1)
            transpose_q(qb, 1)
            start_out(qb, 1)
            return 0

        lax.fori_loop(0, QMAX // 2, pair_body, 0)
        wait_out(0)
        wait_out(1)

    return emb


def kernel(queries, query_lens, weight):
    bsz, qmax = queries.shape
    q2d = queries.reshape(bsz * qmax // 128, 128)
    out5 = _build(bsz)(q2d, query_lens, weight)
    return out5.transpose((2, 4, 0, 1, 3)).reshape(bsz, qmax, D)
